# initial kernel scaffold (unmeasured)
import jax
import jax.numpy as jnp
from jax import lax
from jax.experimental import pallas as pl
from jax.experimental.pallas import tpu as pltpu

N_DEV = 4
N_TOK = 4096
D_IN = 1024
D_OUT = 2048
E_LOCAL = 4
CHUNK = N_TOK // N_DEV
SUB = 512


def kernel(x, router_W, route_idx, expert_W):
    del router_W

    x_bf = x.astype(jnp.bfloat16)
    w_bf = expert_W.astype(jnp.bfloat16)

    def body(x_ref, ri_ref, w_ref, out_ref, comm_ref, send_sems, recv_sems):
        my = lax.axis_index("i")
        left = (my - 1) % N_DEV
        right = (my + 1) % N_DEV

        barrier_sem = pltpu.get_barrier_semaphore()
        for nbr in (left, right):
            pl.semaphore_signal(
                barrier_sem, inc=1,
                device_id=(nbr,), device_id_type=pl.DeviceIdType.MESH,
            )
        pl.semaphore_wait(barrier_sem, 2)

        e0 = my * E_LOCAL
        for t in range(N_TOK // SUB):
            xt = x_ref[pl.ds(t * SUB, SUB), :]
            rit = ri_ref[pl.ds(t * SUB, SUB), :]
            acc = jnp.zeros((SUB, D_OUT), jnp.float32)
            for e in range(E_LOCAL):
                mask = rit == (e0 + e)
                xm = jnp.where(mask, xt, jnp.zeros_like(xt))
                acc = acc + lax.dot_general(
                    xm, w_ref[e],
                    dimension_numbers=(((1,), (0,)), ((), ())),
                    preferred_element_type=jnp.float32,
                )
            out_ref[pl.ds(t * SUB, SUB), :] = acc.astype(jnp.bfloat16)

        for s in range(N_DEV - 1):
            send_c = (my - s) % N_DEV
            recv_c = (my - s - 1) % N_DEV
            slot = s % 2
            rdma = pltpu.make_async_remote_copy(
                src_ref=out_ref.at[pl.ds(send_c * CHUNK, CHUNK), :],
                dst_ref=comm_ref.at[slot],
                send_sem=send_sems.at[s],
                recv_sem=recv_sems.at[s],
                device_id=(right,),
                device_id_type=pl.DeviceIdType.MESH,
            )
            rdma.start()
            rdma.wait()
            cur = out_ref[pl.ds(recv_c * CHUNK, CHUNK), :]
            out_ref[pl.ds(recv_c * CHUNK, CHUNK), :] = cur + comm_ref[slot]

        for s in range(N_DEV - 1):
            c = (my + 1 - s) % N_DEV
            rdma = pltpu.make_async_remote_copy(
                src_ref=out_ref.at[pl.ds(c * CHUNK, CHUNK), :],
                dst_ref=out_ref.at[pl.ds(c * CHUNK, CHUNK), :],
                send_sem=send_sems.at[N_DEV - 1 + s],
                recv_sem=recv_sems.at[N_DEV - 1 + s],
                device_id=(right,),
                device_id_type=pl.DeviceIdType.MESH,
            )
            rdma.start()
            rdma.wait()

    return pl.pallas_call(
        body,
        out_shape=jax.ShapeDtypeStruct((N_TOK, D_OUT), jnp.bfloat16),
        in_specs=[pl.BlockSpec(memory_space=pltpu.VMEM)] * 3,
        out_specs=pl.BlockSpec(memory_space=pltpu.VMEM),
        scratch_shapes=[
            pltpu.VMEM((2, CHUNK, D_OUT), jnp.bfloat16),
            pltpu.SemaphoreType.DMA((2 * (N_DEV - 1),)),
            pltpu.SemaphoreType.DMA((2 * (N_DEV - 1),)),
        ],
        compiler_params=pltpu.CompilerParams(
            collective_id=0,
            vmem_limit_bytes=100 * 1024 * 1024,
        ),
    )(x_bf, route_idx, w_bf)


# baseline (device time: 416941 ns/iter reference)
import jax
import jax.numpy as jnp
from jax import lax
from jax.experimental import pallas as pl
from jax.experimental.pallas import tpu as pltpu

N_DEV = 4
N_TOK = 4096
D_IN = 1024
D_OUT = 2048
E_LOCAL = 4
CHUNK = N_TOK // N_DEV
SUB = 256


def kernel(x, router_W, route_idx, expert_W):
    del router_W

    x_bf = x.astype(jnp.bfloat16)
    w_bf = expert_W.astype(jnp.bfloat16)

    def body(x_ref, ri_ref, w_ref, out_ref, comm_ref, send_sems, recv_sems):
        my = lax.axis_index("i")
        left = (my - 1) % N_DEV
        right = (my + 1) % N_DEV

        barrier_sem = pltpu.get_barrier_semaphore()
        for nbr in (left, right):
            pl.semaphore_signal(
                barrier_sem, inc=1,
                device_id=(nbr,), device_id_type=pl.DeviceIdType.MESH,
            )
        pl.semaphore_wait(barrier_sem, 2)

        e0 = my * E_LOCAL

        def compute_tile(t, carry):
            xt = x_ref[pl.ds(t * SUB, SUB), :]
            rit = ri_ref[pl.ds(t * SUB, SUB), :]
            acc = jnp.zeros((SUB, D_OUT), jnp.float32)
            for e in range(E_LOCAL):
                mask = rit == (e0 + e)
                xm = jnp.where(mask, xt, jnp.zeros_like(xt))
                acc = acc + lax.dot_general(
                    xm, w_ref[e],
                    dimension_numbers=(((1,), (0,)), ((), ())),
                    preferred_element_type=jnp.float32,
                )
            out_ref[pl.ds(t * SUB, SUB), :] = acc.astype(jnp.bfloat16)
            return carry

        lax.fori_loop(0, N_TOK // SUB, compute_tile, 0)

        for s in range(N_DEV - 1):
            send_c = (my - s) % N_DEV
            recv_c = (my - s - 1) % N_DEV
            slot = s % 2
            rdma = pltpu.make_async_remote_copy(
                src_ref=out_ref.at[pl.ds(send_c * CHUNK, CHUNK), :],
                dst_ref=comm_ref.at[slot],
                send_sem=send_sems.at[s],
                recv_sem=recv_sems.at[s],
                device_id=(right,),
                device_id_type=pl.DeviceIdType.MESH,
            )
            rdma.start()
            rdma.wait()
            cur = out_ref[pl.ds(recv_c * CHUNK, CHUNK), :]
            out_ref[pl.ds(recv_c * CHUNK, CHUNK), :] = cur + comm_ref[slot]

        for s in range(N_DEV - 1):
            c = (my + 1 - s) % N_DEV
            rdma = pltpu.make_async_remote_copy(
                src_ref=out_ref.at[pl.ds(c * CHUNK, CHUNK), :],
                dst_ref=out_ref.at[pl.ds(c * CHUNK, CHUNK), :],
                send_sem=send_sems.at[N_DEV - 1 + s],
                recv_sem=recv_sems.at[N_DEV - 1 + s],
                device_id=(right,),
                device_id_type=pl.DeviceIdType.MESH,
            )
            rdma.start()
            rdma.wait()

    return pl.pallas_call(
        body,
        out_shape=jax.ShapeDtypeStruct((N_TOK, D_OUT), jnp.bfloat16),
        in_specs=[pl.BlockSpec(memory_space=pltpu.VMEM)] * 3,
        out_specs=pl.BlockSpec(memory_space=pltpu.VMEM),
        scratch_shapes=[
            pltpu.VMEM((2, CHUNK, D_OUT), jnp.bfloat16),
            pltpu.SemaphoreType.DMA((2 * (N_DEV - 1),)),
            pltpu.SemaphoreType.DMA((2 * (N_DEV - 1),)),
        ],
        compiler_params=pltpu.CompilerParams(
            collective_id=0,
            vmem_limit_bytes=100 * 1024 * 1024,
        ),
    )(x_bf, route_idx, w_bf)


# device time: 240025 ns/iter; 1.7371x vs baseline; 1.7371x over previous
import jax
import jax.numpy as jnp
from jax import lax
from jax.experimental import pallas as pl
from jax.experimental.pallas import tpu as pltpu

N_DEV = 4
N_TOK = 4096
D_IN = 1024
D_OUT = 2048
HALF = D_OUT // 2
E_LOCAL = 4
CHUNK = N_TOK // N_DEV
SUB = 256
N_STEP = N_DEV - 1


def kernel(x, router_W, route_idx, expert_W):
    del router_W

    x_bf = x.astype(jnp.bfloat16)
    w_bf = expert_W.astype(jnp.bfloat16)

    def body(x_ref, ri_ref, w_ref, out_ref, comm_cw, comm_ccw,
             send_sems, recv_sems):
        my = lax.axis_index("i")
        left = (my - 1) % N_DEV
        right = (my + 1) % N_DEV

        barrier_sem = pltpu.get_barrier_semaphore()
        for nbr in (left, right):
            pl.semaphore_signal(
                barrier_sem, inc=1,
                device_id=(nbr,), device_id_type=pl.DeviceIdType.MESH,
            )
        pl.semaphore_wait(barrier_sem, 2)

        e0 = my * E_LOCAL

        def compute_chunk(c):
            base = c * CHUNK

            def tile(t, carry):
                off = base + t * SUB
                xt = x_ref[pl.ds(off, SUB), :]
                rit = ri_ref[pl.ds(off, SUB), :]
                acc = jnp.zeros((SUB, D_OUT), jnp.float32)
                for e in range(E_LOCAL):
                    mask = rit == (e0 + e)
                    xm = jnp.where(mask, xt, jnp.zeros_like(xt))
                    acc = acc + lax.dot_general(
                        xm, w_ref[e],
                        dimension_numbers=(((1,), (0,)), ((), ())),
                        preferred_element_type=jnp.float32,
                    )
                out_ref[pl.ds(off, SUB), :] = acc.astype(jnp.bfloat16)
                return carry

            lax.fori_loop(0, CHUNK // SUB, tile, 0)

        compute_chunk(my)

        compute_after_start = {0: [1, 3], 1: [2], 2: []}
        for s in range(N_STEP):
            slot = s % 2
            cw = pltpu.make_async_remote_copy(
                src_ref=out_ref.at[pl.ds((my - s) % N_DEV * CHUNK, CHUNK),
                                   pl.ds(0, HALF)],
                dst_ref=comm_cw.at[slot],
                send_sem=send_sems.at[0, s],
                recv_sem=recv_sems.at[0, s],
                device_id=(right,),
                device_id_type=pl.DeviceIdType.MESH,
            )
            ccw = pltpu.make_async_remote_copy(
                src_ref=out_ref.at[pl.ds((my + s) % N_DEV * CHUNK, CHUNK),
                                   pl.ds(HALF, HALF)],
                dst_ref=comm_ccw.at[slot],
                send_sem=send_sems.at[1, s],
                recv_sem=recv_sems.at[1, s],
                device_id=(left,),
                device_id_type=pl.DeviceIdType.MESH,
            )
            cw.start()
            ccw.start()
            for off in compute_after_start[s]:
                compute_chunk((my + off) % N_DEV)
            cw.wait()
            ccw.wait()
            rc_cw = (my - s - 1) % N_DEV * CHUNK
            rc_ccw = (my + s + 1) % N_DEV * CHUNK
            out_ref[pl.ds(rc_cw, CHUNK), pl.ds(0, HALF)] = (
                out_ref[pl.ds(rc_cw, CHUNK), pl.ds(0, HALF)] + comm_cw[slot]
            )
            out_ref[pl.ds(rc_ccw, CHUNK), pl.ds(HALF, HALF)] = (
                out_ref[pl.ds(rc_ccw, CHUNK), pl.ds(HALF, HALF)] + comm_ccw[slot]
            )

        for s in range(N_STEP):
            c_cw = (my + 1 - s) % N_DEV * CHUNK
            c_ccw = (my - 1 + s) % N_DEV * CHUNK
            cw = pltpu.make_async_remote_copy(
                src_ref=out_ref.at[pl.ds(c_cw, CHUNK), pl.ds(0, HALF)],
                dst_ref=out_ref.at[pl.ds(c_cw, CHUNK), pl.ds(0, HALF)],
                send_sem=send_sems.at[0, N_STEP + s],
                recv_sem=recv_sems.at[0, N_STEP + s],
                device_id=(right,),
                device_id_type=pl.DeviceIdType.MESH,
            )
            ccw = pltpu.make_async_remote_copy(
                src_ref=out_ref.at[pl.ds(c_ccw, CHUNK), pl.ds(HALF, HALF)],
                dst_ref=out_ref.at[pl.ds(c_ccw, CHUNK), pl.ds(HALF, HALF)],
                send_sem=send_sems.at[1, N_STEP + s],
                recv_sem=recv_sems.at[1, N_STEP + s],
                device_id=(left,),
                device_id_type=pl.DeviceIdType.MESH,
            )
            cw.start()
            ccw.start()
            cw.wait()
            ccw.wait()

    return pl.pallas_call(
        body,
        out_shape=jax.ShapeDtypeStruct((N_TOK, D_OUT), jnp.bfloat16),
        in_specs=[pl.BlockSpec(memory_space=pltpu.VMEM)] * 3,
        out_specs=pl.BlockSpec(memory_space=pltpu.VMEM),
        scratch_shapes=[
            pltpu.VMEM((2, CHUNK, HALF), jnp.bfloat16),
            pltpu.VMEM((2, CHUNK, HALF), jnp.bfloat16),
            pltpu.SemaphoreType.DMA((2, 2 * N_STEP)),
            pltpu.SemaphoreType.DMA((2, 2 * N_STEP)),
        ],
        compiler_params=pltpu.CompilerParams(
            collective_id=0,
            vmem_limit_bytes=100 * 1024 * 1024,
        ),
    )(x_bf, route_idx, w_bf)
